# Initial kernel scaffold; baseline (speedup 1.0000x reference)
import jax
import jax.numpy as jnp
from jax.experimental import pallas as pl


def kernel(x, edge_index, W1, as1, ad1, b1, W2, as2, ad2, b2, W3, as3, ad3, b3, fcW, fcb):
    raise NotImplementedError("write your pallas kernel here")


# speedup vs baseline: 1.8168x; 1.8168x over previous
import jax
import jax.numpy as jnp
from jax.experimental import pallas as pl
from jax.experimental.pallas import tpu as pltpu

N_NODES = 10000
NP = 10240
HID = 256
NHEADS = 3
H3 = NHEADS * HID
EP = 172032
PAD_IDX = 10208

EB_G = 512
NB_E = 2048
NB_D = 1024
EB_S = 2048
NB_S = 512
LRELU = 0.2


def _proj_body(x_ref, w_ref, asrc_ref, adst_ref, h_ref, sa_ref, sd_ref):
    h = jnp.dot(x_ref[:], w_ref[:], preferred_element_type=jnp.float32)
    h_ref[:] = h.astype(jnp.bfloat16)
    sa_ref[:] = jnp.dot(h, asrc_ref[:], preferred_element_type=jnp.float32)
    sd_ref[:] = jnp.dot(h, adst_ref[:], preferred_element_type=jnp.float32)


def _edge_e_body(src_ref, dst_ref, sa_ref, sd_ref, ee_ref, acc):
    j = pl.program_id(1)

    @pl.when(j == 0)
    def _():
        acc[:] = jnp.zeros_like(acc)

    base = j * NB_E
    it = jax.lax.broadcasted_iota(jnp.int32, (EB_G, NB_E), 1) + base
    ohs = (src_ref[:] == it).astype(jnp.bfloat16)
    ohd = (dst_ref[:] == it).astype(jnp.bfloat16)
    acc[:] += (
        jnp.dot(ohs, sa_ref[:].astype(jnp.bfloat16),
                preferred_element_type=jnp.float32)
        + jnp.dot(ohd, sd_ref[:].astype(jnp.bfloat16),
                  preferred_element_type=jnp.float32))

    @pl.when(j == pl.num_programs(1) - 1)
    def _():
        e = acc[:]
        e = jnp.where(e > 0, e, LRELU * e)
        ee_ref[:] = jnp.exp(e)


def _denom_body(dstr_ref, ee_ref, den_ref):
    j = pl.program_id(1)

    @pl.when(j == 0)
    def _():
        den_ref[:] = jnp.zeros_like(den_ref)

    base = pl.program_id(0) * NB_S
    it = jax.lax.broadcasted_iota(jnp.int32, (NB_S, EB_S), 0) + base
    oht = (it == dstr_ref[:]).astype(jnp.bfloat16)
    den_ref[:] += jnp.dot(oht, ee_ref[:].astype(jnp.bfloat16),
                          preferred_element_type=jnp.float32)


def _msg_body(src_ref, dst_ref, ee_ref, den_ref, h_ref, msg_ref, hs, ds):
    j = pl.program_id(1)

    @pl.when(j == 0)
    def _():
        hs[:] = jnp.zeros_like(hs)
        ds[:] = jnp.zeros_like(ds)

    base = j * NB_D
    it = jax.lax.broadcasted_iota(jnp.int32, (EB_G, NB_D), 1) + base
    ohs = (src_ref[:] == it).astype(jnp.bfloat16)
    ohd = (dst_ref[:] == it).astype(jnp.bfloat16)
    hs[:] += jnp.dot(ohs, h_ref[:], preferred_element_type=jnp.float32)
    ds[:] += jnp.dot(ohd, den_ref[:].astype(jnp.bfloat16),
                     preferred_element_type=jnp.float32)

    @pl.when(j == pl.num_programs(1) - 1)
    def _():
        alpha = ee_ref[:] / (ds[:] + 1e-16)
        for hh in range(NHEADS):
            msg_ref[:, hh * HID:(hh + 1) * HID] = (
                hs[:, hh * HID:(hh + 1) * HID]
                * alpha[:, hh:hh + 1]).astype(jnp.bfloat16)


def _agg_body(dstr_ref, msg_ref, agg_ref):
    j = pl.program_id(1)

    @pl.when(j == 0)
    def _():
        agg_ref[:] = jnp.zeros_like(agg_ref)

    base = pl.program_id(0) * NB_S
    it = jax.lax.broadcasted_iota(jnp.int32, (NB_S, EB_S), 0) + base
    oht = (it == dstr_ref[:]).astype(jnp.bfloat16)
    agg_ref[:] += jnp.dot(oht, msg_ref[:], preferred_element_type=jnp.float32)


def _headmean_body(agg_ref, b_ref, y_ref):
    m = (agg_ref[:, 0:HID] + agg_ref[:, HID:2 * HID]
         + agg_ref[:, 2 * HID:3 * HID]) * (1.0 / 3.0) + b_ref[:]
    y_ref[:] = jnp.where(m > 0, m, jnp.exp(jnp.minimum(m, 0.0)) - 1.0)


def _fc_body(y_ref, w_ref, b_ref, o_ref):
    o_ref[:] = jnp.dot(y_ref[:], w_ref[:],
                       preferred_element_type=jnp.float32) + b_ref[:]


def _gat_layer(xp, src_col, dst_col, dst_row, Wp, Amat_s, Amat_d, b):
    K = xp.shape[1]
    NE_G = EP // EB_G
    NE_S = EP // EB_S

    h, sa, sd = pl.pallas_call(
        _proj_body,
        grid=(NP // 256,),
        in_specs=[
            pl.BlockSpec((256, K), lambda i: (i, 0)),
            pl.BlockSpec((K, H3), lambda i: (0, 0)),
            pl.BlockSpec((H3, 128), lambda i: (0, 0)),
            pl.BlockSpec((H3, 128), lambda i: (0, 0)),
        ],
        out_specs=[
            pl.BlockSpec((256, H3), lambda i: (i, 0)),
            pl.BlockSpec((256, 128), lambda i: (i, 0)),
            pl.BlockSpec((256, 128), lambda i: (i, 0)),
        ],
        out_shape=[
            jax.ShapeDtypeStruct((NP, H3), jnp.bfloat16),
            jax.ShapeDtypeStruct((NP, 128), jnp.float32),
            jax.ShapeDtypeStruct((NP, 128), jnp.float32),
        ],
    )(xp, Wp, Amat_s, Amat_d)

    ee = pl.pallas_call(
        _edge_e_body,
        grid=(NE_G, NP // NB_E),
        in_specs=[
            pl.BlockSpec((EB_G, 1), lambda i, j: (i, 0)),
            pl.BlockSpec((EB_G, 1), lambda i, j: (i, 0)),
            pl.BlockSpec((NB_E, 128), lambda i, j: (j, 0)),
            pl.BlockSpec((NB_E, 128), lambda i, j: (j, 0)),
        ],
        out_specs=pl.BlockSpec((EB_G, 128), lambda i, j: (i, 0)),
        out_shape=jax.ShapeDtypeStruct((EP, 128), jnp.float32),
        scratch_shapes=[pltpu.VMEM((EB_G, 128), jnp.float32)],
    )(src_col, dst_col, sa, sd)

    den = pl.pallas_call(
        _denom_body,
        grid=(NP // NB_S, NE_S),
        in_specs=[
            pl.BlockSpec((1, EB_S), lambda i, j: (0, j)),
            pl.BlockSpec((EB_S, 128), lambda i, j: (j, 0)),
        ],
        out_specs=pl.BlockSpec((NB_S, 128), lambda i, j: (i, 0)),
        out_shape=jax.ShapeDtypeStruct((NP, 128), jnp.float32),
    )(dst_row, ee)

    msg = pl.pallas_call(
        _msg_body,
        grid=(NE_G, NP // NB_D),
        in_specs=[
            pl.BlockSpec((EB_G, 1), lambda i, j: (i, 0)),
            pl.BlockSpec((EB_G, 1), lambda i, j: (i, 0)),
            pl.BlockSpec((EB_G, 128), lambda i, j: (i, 0)),
            pl.BlockSpec((NB_D, 128), lambda i, j: (j, 0)),
            pl.BlockSpec((NB_D, H3), lambda i, j: (j, 0)),
        ],
        out_specs=pl.BlockSpec((EB_G, H3), lambda i, j: (i, 0)),
        out_shape=jax.ShapeDtypeStruct((EP, H3), jnp.bfloat16),
        scratch_shapes=[
            pltpu.VMEM((EB_G, H3), jnp.float32),
            pltpu.VMEM((EB_G, 128), jnp.float32),
        ],
    )(src_col, dst_col, ee, den, h)

    agg = pl.pallas_call(
        _agg_body,
        grid=(NP // NB_S, NE_S),
        in_specs=[
            pl.BlockSpec((1, EB_S), lambda i, j: (0, j)),
            pl.BlockSpec((EB_S, H3), lambda i, j: (j, 0)),
        ],
        out_specs=pl.BlockSpec((NB_S, H3), lambda i, j: (i, 0)),
        out_shape=jax.ShapeDtypeStruct((NP, H3), jnp.float32),
    )(dst_row, msg)

    y = pl.pallas_call(
        _headmean_body,
        grid=(NP // NB_S,),
        in_specs=[
            pl.BlockSpec((NB_S, H3), lambda i: (i, 0)),
            pl.BlockSpec((1, HID), lambda i: (0, 0)),
        ],
        out_specs=pl.BlockSpec((NB_S, HID), lambda i: (i, 0)),
        out_shape=jax.ShapeDtypeStruct((NP, HID), jnp.float32),
    )(agg, b.reshape(1, HID))

    return y


def _amat(a):
    m = jnp.zeros((H3, 128), jnp.float32)
    for hh in range(NHEADS):
        m = m.at[hh * HID:(hh + 1) * HID, hh].set(a[0, hh])
    return m


def kernel(x, edge_index, W1, as1, ad1, b1, W2, as2, ad2, b2,
           W3, as3, ad3, b3, fcW, fcb):
    loop = jnp.arange(N_NODES, dtype=edge_index.dtype)
    src = jnp.concatenate([edge_index[0], loop])
    dst = jnp.concatenate([edge_index[1], loop])
    npad = EP - src.shape[0]
    src = jnp.concatenate([src, jnp.full((npad,), PAD_IDX, src.dtype)])
    dst = jnp.concatenate([dst, jnp.full((npad,), PAD_IDX, dst.dtype)])
    src_col = src.reshape(EP, 1)
    dst_col = dst.reshape(EP, 1)
    dst_row = dst.reshape(1, EP)

    xp = jnp.zeros((NP, 64), jnp.float32)
    xp = xp.at[:N_NODES, :x.shape[1]].set(x)
    W1p = jnp.zeros((64, H3), jnp.float32).at[:W1.shape[0]].set(W1)

    y = _gat_layer(xp, src_col, dst_col, dst_row, W1p,
                   _amat(as1), _amat(ad1), b1)
    y = _gat_layer(y, src_col, dst_col, dst_row, W2,
                   _amat(as2), _amat(ad2), b2)
    y = _gat_layer(y, src_col, dst_col, dst_row, W3,
                   _amat(as3), _amat(ad3), b3)

    fcWp = jnp.zeros((HID, 128), jnp.float32).at[:, :fcW.shape[1]].set(fcW)
    fcbp = jnp.zeros((1, 128), jnp.float32).at[0, :fcb.shape[0]].set(fcb)
    out = pl.pallas_call(
        _fc_body,
        grid=(NP // NB_S,),
        in_specs=[
            pl.BlockSpec((NB_S, HID), lambda i: (i, 0)),
            pl.BlockSpec((HID, 128), lambda i: (0, 0)),
            pl.BlockSpec((1, 128), lambda i: (0, 0)),
        ],
        out_specs=pl.BlockSpec((NB_S, 128), lambda i: (i, 0)),
        out_shape=jax.ShapeDtypeStruct((NP, 128), jnp.float32),
    )(y, fcWp, fcbp)
    return out[:N_NODES, :121]


# speedup vs baseline: 2.2353x; 1.2304x over previous
import jax
import jax.numpy as jnp
from jax.experimental import pallas as pl
from jax.experimental.pallas import tpu as pltpu

N_NODES = 10000
NP = 10240
HID = 256
NHEADS = 3
H3 = NHEADS * HID
EP = 172032
PAD_IDX = 10208

EB_G = 1024
NB_E = 2048
NB_D = 2048
EB_S = 4096
NB_S = 512
LRELU = 0.2


def _proj_body(x_ref, w_ref, asrc_ref, adst_ref, h_ref, sa_ref, sd_ref):
    h = jnp.dot(x_ref[:], w_ref[:], preferred_element_type=jnp.float32)
    h_ref[:] = h.astype(jnp.bfloat16)
    sa_ref[:] = jnp.dot(h, asrc_ref[:], preferred_element_type=jnp.float32)
    sd_ref[:] = jnp.dot(h, adst_ref[:], preferred_element_type=jnp.float32)


def _edge_e_body(src_ref, dst_ref, sa_ref, sd_ref, ee_ref, acc):
    j = pl.program_id(1)

    @pl.when(j == 0)
    def _():
        acc[:] = jnp.zeros_like(acc)

    base = j * NB_E
    it = jax.lax.broadcasted_iota(jnp.int32, (EB_G, NB_E), 1) + base
    ohs = (src_ref[:] == it).astype(jnp.bfloat16)
    ohd = (dst_ref[:] == it).astype(jnp.bfloat16)
    acc[:] += (
        jnp.dot(ohs, sa_ref[:].astype(jnp.bfloat16),
                preferred_element_type=jnp.float32)
        + jnp.dot(ohd, sd_ref[:].astype(jnp.bfloat16),
                  preferred_element_type=jnp.float32))

    @pl.when(j == pl.num_programs(1) - 1)
    def _():
        e = acc[:]
        e = jnp.where(e > 0, e, LRELU * e)
        ee_ref[:] = jnp.exp(e)


def _denom_body(dstr_ref, ee_ref, den_ref):
    j = pl.program_id(1)

    @pl.when(j == 0)
    def _():
        den_ref[:] = jnp.zeros_like(den_ref)

    base = pl.program_id(0) * NB_S
    it = jax.lax.broadcasted_iota(jnp.int32, (NB_S, EB_S), 0) + base
    oht = (it == dstr_ref[:]).astype(jnp.bfloat16)
    den_ref[:] += jnp.dot(oht, ee_ref[:].astype(jnp.bfloat16),
                          preferred_element_type=jnp.float32)


def _msg_body(src_ref, dst_ref, ee_ref, den_ref, h_ref, msg_ref, hs, ds):
    j = pl.program_id(1)

    @pl.when(j == 0)
    def _():
        hs[:] = jnp.zeros_like(hs)
        ds[:] = jnp.zeros_like(ds)

    base = j * NB_D
    it = jax.lax.broadcasted_iota(jnp.int32, (EB_G, NB_D), 1) + base
    ohs = (src_ref[:] == it).astype(jnp.bfloat16)
    ohd = (dst_ref[:] == it).astype(jnp.bfloat16)
    hs[:] += jnp.dot(ohs, h_ref[:], preferred_element_type=jnp.float32)
    ds[:] += jnp.dot(ohd, den_ref[:].astype(jnp.bfloat16),
                     preferred_element_type=jnp.float32)

    @pl.when(j == pl.num_programs(1) - 1)
    def _():
        alpha = ee_ref[:] / (ds[:] + 1e-16)
        for hh in range(NHEADS):
            msg_ref[:, hh * HID:(hh + 1) * HID] = (
                hs[:, hh * HID:(hh + 1) * HID]
                * alpha[:, hh:hh + 1]).astype(jnp.bfloat16)


def _agg_body(dstr_ref, msg_ref, agg_ref):
    j = pl.program_id(1)

    @pl.when(j == 0)
    def _():
        agg_ref[:] = jnp.zeros_like(agg_ref)

    base = pl.program_id(0) * NB_S
    it = jax.lax.broadcasted_iota(jnp.int32, (NB_S, EB_S), 0) + base
    oht = (it == dstr_ref[:]).astype(jnp.bfloat16)
    agg_ref[:] += jnp.dot(oht, msg_ref[:], preferred_element_type=jnp.float32)


def _headmean_body(agg_ref, b_ref, y_ref):
    m = (agg_ref[:, 0:HID] + agg_ref[:, HID:2 * HID]
         + agg_ref[:, 2 * HID:3 * HID]) * (1.0 / 3.0) + b_ref[:]
    y_ref[:] = jnp.where(m > 0, m, jnp.exp(jnp.minimum(m, 0.0)) - 1.0)


def _fc_body(y_ref, w_ref, b_ref, o_ref):
    o_ref[:] = jnp.dot(y_ref[:], w_ref[:],
                       preferred_element_type=jnp.float32) + b_ref[:]


def _gat_layer(xp, src_col, dst_col, dst_row, Wp, Amat_s, Amat_d, b):
    K = xp.shape[1]
    NE_G = EP // EB_G
    NE_S = EP // EB_S

    h, sa, sd = pl.pallas_call(
        _proj_body,
        grid=(NP // 256,),
        in_specs=[
            pl.BlockSpec((256, K), lambda i: (i, 0)),
            pl.BlockSpec((K, H3), lambda i: (0, 0)),
            pl.BlockSpec((H3, 128), lambda i: (0, 0)),
            pl.BlockSpec((H3, 128), lambda i: (0, 0)),
        ],
        out_specs=[
            pl.BlockSpec((256, H3), lambda i: (i, 0)),
            pl.BlockSpec((256, 128), lambda i: (i, 0)),
            pl.BlockSpec((256, 128), lambda i: (i, 0)),
        ],
        out_shape=[
            jax.ShapeDtypeStruct((NP, H3), jnp.bfloat16),
            jax.ShapeDtypeStruct((NP, 128), jnp.float32),
            jax.ShapeDtypeStruct((NP, 128), jnp.float32),
        ],
    )(xp, Wp, Amat_s, Amat_d)

    ee = pl.pallas_call(
        _edge_e_body,
        grid=(NE_G, NP // NB_E),
        in_specs=[
            pl.BlockSpec((EB_G, 1), lambda i, j: (i, 0)),
            pl.BlockSpec((EB_G, 1), lambda i, j: (i, 0)),
            pl.BlockSpec((NB_E, 128), lambda i, j: (j, 0)),
            pl.BlockSpec((NB_E, 128), lambda i, j: (j, 0)),
        ],
        out_specs=pl.BlockSpec((EB_G, 128), lambda i, j: (i, 0)),
        out_shape=jax.ShapeDtypeStruct((EP, 128), jnp.float32),
        scratch_shapes=[pltpu.VMEM((EB_G, 128), jnp.float32)],
    )(src_col, dst_col, sa, sd)

    den = pl.pallas_call(
        _denom_body,
        grid=(NP // NB_S, NE_S),
        in_specs=[
            pl.BlockSpec((1, EB_S), lambda i, j: (0, j)),
            pl.BlockSpec((EB_S, 128), lambda i, j: (j, 0)),
        ],
        out_specs=pl.BlockSpec((NB_S, 128), lambda i, j: (i, 0)),
        out_shape=jax.ShapeDtypeStruct((NP, 128), jnp.float32),
    )(dst_row, ee)

    msg = pl.pallas_call(
        _msg_body,
        grid=(NE_G, NP // NB_D),
        in_specs=[
            pl.BlockSpec((EB_G, 1), lambda i, j: (i, 0)),
            pl.BlockSpec((EB_G, 1), lambda i, j: (i, 0)),
            pl.BlockSpec((EB_G, 128), lambda i, j: (i, 0)),
            pl.BlockSpec((NB_D, 128), lambda i, j: (j, 0)),
            pl.BlockSpec((NB_D, H3), lambda i, j: (j, 0)),
        ],
        out_specs=pl.BlockSpec((EB_G, H3), lambda i, j: (i, 0)),
        out_shape=jax.ShapeDtypeStruct((EP, H3), jnp.bfloat16),
        scratch_shapes=[
            pltpu.VMEM((EB_G, H3), jnp.float32),
            pltpu.VMEM((EB_G, 128), jnp.float32),
        ],
    )(src_col, dst_col, ee, den, h)

    agg = pl.pallas_call(
        _agg_body,
        grid=(NP // NB_S, NE_S),
        in_specs=[
            pl.BlockSpec((1, EB_S), lambda i, j: (0, j)),
            pl.BlockSpec((EB_S, H3), lambda i, j: (j, 0)),
        ],
        out_specs=pl.BlockSpec((NB_S, H3), lambda i, j: (i, 0)),
        out_shape=jax.ShapeDtypeStruct((NP, H3), jnp.float32),
    )(dst_row, msg)

    y = pl.pallas_call(
        _headmean_body,
        grid=(NP // NB_S,),
        in_specs=[
            pl.BlockSpec((NB_S, H3), lambda i: (i, 0)),
            pl.BlockSpec((1, HID), lambda i: (0, 0)),
        ],
        out_specs=pl.BlockSpec((NB_S, HID), lambda i: (i, 0)),
        out_shape=jax.ShapeDtypeStruct((NP, HID), jnp.float32),
    )(agg, b.reshape(1, HID))

    return y


def _amat(a):
    m = jnp.zeros((H3, 128), jnp.float32)
    for hh in range(NHEADS):
        m = m.at[hh * HID:(hh + 1) * HID, hh].set(a[0, hh])
    return m


def kernel(x, edge_index, W1, as1, ad1, b1, W2, as2, ad2, b2,
           W3, as3, ad3, b3, fcW, fcb):
    loop = jnp.arange(N_NODES, dtype=edge_index.dtype)
    src = jnp.concatenate([edge_index[0], loop])
    dst = jnp.concatenate([edge_index[1], loop])
    npad = EP - src.shape[0]
    src = jnp.concatenate([src, jnp.full((npad,), PAD_IDX, src.dtype)])
    dst = jnp.concatenate([dst, jnp.full((npad,), PAD_IDX, dst.dtype)])
    src_col = src.reshape(EP, 1)
    dst_col = dst.reshape(EP, 1)
    dst_row = dst.reshape(1, EP)

    xp = jnp.zeros((NP, 64), jnp.float32)
    xp = xp.at[:N_NODES, :x.shape[1]].set(x)
    W1p = jnp.zeros((64, H3), jnp.float32).at[:W1.shape[0]].set(W1)

    y = _gat_layer(xp, src_col, dst_col, dst_row, W1p,
                   _amat(as1), _amat(ad1), b1)
    y = _gat_layer(y, src_col, dst_col, dst_row, W2,
                   _amat(as2), _amat(ad2), b2)
    y = _gat_layer(y, src_col, dst_col, dst_row, W3,
                   _amat(as3), _amat(ad3), b3)

    fcWp = jnp.zeros((HID, 128), jnp.float32).at[:, :fcW.shape[1]].set(fcW)
    fcbp = jnp.zeros((1, 128), jnp.float32).at[0, :fcb.shape[0]].set(fcb)
    out = pl.pallas_call(
        _fc_body,
        grid=(NP // NB_S,),
        in_specs=[
            pl.BlockSpec((NB_S, HID), lambda i: (i, 0)),
            pl.BlockSpec((HID, 128), lambda i: (0, 0)),
            pl.BlockSpec((1, 128), lambda i: (0, 0)),
        ],
        out_specs=pl.BlockSpec((NB_S, 128), lambda i: (i, 0)),
        out_shape=jax.ShapeDtypeStruct((NP, 128), jnp.float32),
    )(y, fcWp, fcbp)
    return out[:N_NODES, :121]


# speedup vs baseline: 2.3066x; 1.0319x over previous
import jax
import jax.numpy as jnp
from jax.experimental import pallas as pl
from jax.experimental.pallas import tpu as pltpu

N_NODES = 10000
NP = 10240
HID = 256
NHEADS = 3
H3 = NHEADS * HID
EP = 172032
PAD_IDX = 10208

EB_G = 2048
NB_E = 2048
NB_D = 2048
EB_S = 4096
NB_S = 1024
LRELU = 0.2


def _proj_body(x_ref, w_ref, asrc_ref, adst_ref, h_ref, sa_ref, sd_ref):
    h = jnp.dot(x_ref[:], w_ref[:], preferred_element_type=jnp.float32)
    h_ref[:] = h.astype(jnp.bfloat16)
    sa_ref[:] = jnp.dot(h, asrc_ref[:], preferred_element_type=jnp.float32)
    sd_ref[:] = jnp.dot(h, adst_ref[:], preferred_element_type=jnp.float32)


def _edge_e_body(src_ref, dst_ref, sa_ref, sd_ref, ee_ref, acc):
    j = pl.program_id(1)

    @pl.when(j == 0)
    def _():
        acc[:] = jnp.zeros_like(acc)

    base = j * NB_E
    it = jax.lax.broadcasted_iota(jnp.int32, (EB_G, NB_E), 1) + base
    ohs = (src_ref[:] == it).astype(jnp.bfloat16)
    ohd = (dst_ref[:] == it).astype(jnp.bfloat16)
    acc[:] += (
        jnp.dot(ohs, sa_ref[:].astype(jnp.bfloat16),
                preferred_element_type=jnp.float32)
        + jnp.dot(ohd, sd_ref[:].astype(jnp.bfloat16),
                  preferred_element_type=jnp.float32))

    @pl.when(j == pl.num_programs(1) - 1)
    def _():
        e = acc[:]
        e = jnp.where(e > 0, e, LRELU * e)
        ee_ref[:] = jnp.exp(e)


def _denom_body(dstr_ref, ee_ref, den_ref):
    j = pl.program_id(1)

    @pl.when(j == 0)
    def _():
        den_ref[:] = jnp.zeros_like(den_ref)

    base = pl.program_id(0) * NB_S
    it = jax.lax.broadcasted_iota(jnp.int32, (NB_S, EB_S), 0) + base
    oht = (it == dstr_ref[:]).astype(jnp.bfloat16)
    den_ref[:] += jnp.dot(oht, ee_ref[:].astype(jnp.bfloat16),
                          preferred_element_type=jnp.float32)


def _msg_body(src_ref, dst_ref, ee_ref, den_ref, h_ref, msg_ref, hs, ds):
    j = pl.program_id(1)

    @pl.when(j == 0)
    def _():
        hs[:] = jnp.zeros_like(hs)
        ds[:] = jnp.zeros_like(ds)

    base = j * NB_D
    it = jax.lax.broadcasted_iota(jnp.int32, (EB_G, NB_D), 1) + base
    ohs = (src_ref[:] == it).astype(jnp.bfloat16)
    ohd = (dst_ref[:] == it).astype(jnp.bfloat16)
    hs[:] += jnp.dot(ohs, h_ref[:], preferred_element_type=jnp.float32)
    ds[:] += jnp.dot(ohd, den_ref[:].astype(jnp.bfloat16),
                     preferred_element_type=jnp.float32)

    @pl.when(j == pl.num_programs(1) - 1)
    def _():
        alpha = ee_ref[:] / (ds[:] + 1e-16)
        for hh in range(NHEADS):
            msg_ref[:, hh * HID:(hh + 1) * HID] = (
                hs[:, hh * HID:(hh + 1) * HID]
                * alpha[:, hh:hh + 1]).astype(jnp.bfloat16)


def _agg_body(dstr_ref, msg_ref, agg_ref):
    j = pl.program_id(1)

    @pl.when(j == 0)
    def _():
        agg_ref[:] = jnp.zeros_like(agg_ref)

    base = pl.program_id(0) * NB_S
    it = jax.lax.broadcasted_iota(jnp.int32, (NB_S, EB_S), 0) + base
    oht = (it == dstr_ref[:]).astype(jnp.bfloat16)
    agg_ref[:] += jnp.dot(oht, msg_ref[:], preferred_element_type=jnp.float32)


def _headmean_body(agg_ref, b_ref, y_ref):
    m = (agg_ref[:, 0:HID] + agg_ref[:, HID:2 * HID]
         + agg_ref[:, 2 * HID:3 * HID]) * (1.0 / 3.0) + b_ref[:]
    y_ref[:] = jnp.where(m > 0, m, jnp.exp(jnp.minimum(m, 0.0)) - 1.0)


def _fc_body(y_ref, w_ref, b_ref, o_ref):
    o_ref[:] = jnp.dot(y_ref[:], w_ref[:],
                       preferred_element_type=jnp.float32) + b_ref[:]


def _gat_layer(xp, src_col, dst_col, dst_row, Wp, Amat_s, Amat_d, b):
    K = xp.shape[1]
    NE_G = EP // EB_G
    NE_S = EP // EB_S

    h, sa, sd = pl.pallas_call(
        _proj_body,
        grid=(NP // 256,),
        in_specs=[
            pl.BlockSpec((256, K), lambda i: (i, 0)),
            pl.BlockSpec((K, H3), lambda i: (0, 0)),
            pl.BlockSpec((H3, 128), lambda i: (0, 0)),
            pl.BlockSpec((H3, 128), lambda i: (0, 0)),
        ],
        out_specs=[
            pl.BlockSpec((256, H3), lambda i: (i, 0)),
            pl.BlockSpec((256, 128), lambda i: (i, 0)),
            pl.BlockSpec((256, 128), lambda i: (i, 0)),
        ],
        out_shape=[
            jax.ShapeDtypeStruct((NP, H3), jnp.bfloat16),
            jax.ShapeDtypeStruct((NP, 128), jnp.float32),
            jax.ShapeDtypeStruct((NP, 128), jnp.float32),
        ],
    )(xp, Wp, Amat_s, Amat_d)

    ee = pl.pallas_call(
        _edge_e_body,
        grid=(NE_G, NP // NB_E),
        in_specs=[
            pl.BlockSpec((EB_G, 1), lambda i, j: (i, 0)),
            pl.BlockSpec((EB_G, 1), lambda i, j: (i, 0)),
            pl.BlockSpec((NB_E, 128), lambda i, j: (j, 0)),
            pl.BlockSpec((NB_E, 128), lambda i, j: (j, 0)),
        ],
        out_specs=pl.BlockSpec((EB_G, 128), lambda i, j: (i, 0)),
        out_shape=jax.ShapeDtypeStruct((EP, 128), jnp.float32),
        scratch_shapes=[pltpu.VMEM((EB_G, 128), jnp.float32)],
    )(src_col, dst_col, sa, sd)

    den = pl.pallas_call(
        _denom_body,
        grid=(NP // NB_S, NE_S),
        in_specs=[
            pl.BlockSpec((1, EB_S), lambda i, j: (0, j)),
            pl.BlockSpec((EB_S, 128), lambda i, j: (j, 0)),
        ],
        out_specs=pl.BlockSpec((NB_S, 128), lambda i, j: (i, 0)),
        out_shape=jax.ShapeDtypeStruct((NP, 128), jnp.float32),
    )(dst_row, ee)

    msg = pl.pallas_call(
        _msg_body,
        grid=(NE_G, NP // NB_D),
        in_specs=[
            pl.BlockSpec((EB_G, 1), lambda i, j: (i, 0)),
            pl.BlockSpec((EB_G, 1), lambda i, j: (i, 0)),
            pl.BlockSpec((EB_G, 128), lambda i, j: (i, 0)),
            pl.BlockSpec((NB_D, 128), lambda i, j: (j, 0)),
            pl.BlockSpec((NB_D, H3), lambda i, j: (j, 0)),
        ],
        out_specs=pl.BlockSpec((EB_G, H3), lambda i, j: (i, 0)),
        out_shape=jax.ShapeDtypeStruct((EP, H3), jnp.bfloat16),
        scratch_shapes=[
            pltpu.VMEM((EB_G, H3), jnp.float32),
            pltpu.VMEM((EB_G, 128), jnp.float32),
        ],
    )(src_col, dst_col, ee, den, h)

    agg = pl.pallas_call(
        _agg_body,
        grid=(NP // NB_S, NE_S),
        in_specs=[
            pl.BlockSpec((1, EB_S), lambda i, j: (0, j)),
            pl.BlockSpec((EB_S, H3), lambda i, j: (j, 0)),
        ],
        out_specs=pl.BlockSpec((NB_S, H3), lambda i, j: (i, 0)),
        out_shape=jax.ShapeDtypeStruct((NP, H3), jnp.float32),
    )(dst_row, msg)

    y = pl.pallas_call(
        _headmean_body,
        grid=(NP // NB_S,),
        in_specs=[
            pl.BlockSpec((NB_S, H3), lambda i: (i, 0)),
            pl.BlockSpec((1, HID), lambda i: (0, 0)),
        ],
        out_specs=pl.BlockSpec((NB_S, HID), lambda i: (i, 0)),
        out_shape=jax.ShapeDtypeStruct((NP, HID), jnp.float32),
    )(agg, b.reshape(1, HID))

    return y


def _amat(a):
    m = jnp.zeros((H3, 128), jnp.float32)
    for hh in range(NHEADS):
        m = m.at[hh * HID:(hh + 1) * HID, hh].set(a[0, hh])
    return m


def kernel(x, edge_index, W1, as1, ad1, b1, W2, as2, ad2, b2,
           W3, as3, ad3, b3, fcW, fcb):
    loop = jnp.arange(N_NODES, dtype=edge_index.dtype)
    src = jnp.concatenate([edge_index[0], loop])
    dst = jnp.concatenate([edge_index[1], loop])
    npad = EP - src.shape[0]
    src = jnp.concatenate([src, jnp.full((npad,), PAD_IDX, src.dtype)])
    dst = jnp.concatenate([dst, jnp.full((npad,), PAD_IDX, dst.dtype)])
    src_col = src.reshape(EP, 1)
    dst_col = dst.reshape(EP, 1)
    dst_row = dst.reshape(1, EP)

    xp = jnp.zeros((NP, 64), jnp.float32)
    xp = xp.at[:N_NODES, :x.shape[1]].set(x)
    W1p = jnp.zeros((64, H3), jnp.float32).at[:W1.shape[0]].set(W1)

    y = _gat_layer(xp, src_col, dst_col, dst_row, W1p,
                   _amat(as1), _amat(ad1), b1)
    y = _gat_layer(y, src_col, dst_col, dst_row, W2,
                   _amat(as2), _amat(ad2), b2)
    y = _gat_layer(y, src_col, dst_col, dst_row, W3,
                   _amat(as3), _amat(ad3), b3)

    fcWp = jnp.zeros((HID, 128), jnp.float32).at[:, :fcW.shape[1]].set(fcW)
    fcbp = jnp.zeros((1, 128), jnp.float32).at[0, :fcb.shape[0]].set(fcb)
    out = pl.pallas_call(
        _fc_body,
        grid=(NP // NB_S,),
        in_specs=[
            pl.BlockSpec((NB_S, HID), lambda i: (i, 0)),
            pl.BlockSpec((HID, 128), lambda i: (0, 0)),
            pl.BlockSpec((1, 128), lambda i: (0, 0)),
        ],
        out_specs=pl.BlockSpec((NB_S, 128), lambda i: (i, 0)),
        out_shape=jax.ShapeDtypeStruct((NP, 128), jnp.float32),
    )(y, fcWp, fcbp)
    return out[:N_NODES, :121]
